# Initial kernel scaffold; baseline (speedup 1.0000x reference)
import jax
import jax.numpy as jnp
from jax.experimental import pallas as pl


def kernel(X, params, edge_index, com_div):
    raise NotImplementedError("write your pallas kernel here")


# speedup vs baseline: 1.0012x; 1.0012x over previous
import jax
import jax.numpy as jnp
from jax.experimental import pallas as pl
from jax.experimental.pallas import tpu as pltpu

COM = 64


def _seg_mean(msgs, idx, n):
    s = jax.ops.segment_sum(msgs, idx, num_segments=n)
    c = jax.ops.segment_sum(jnp.ones((idx.shape[0], 1), msgs.dtype), idx, num_segments=n)
    return s / jnp.maximum(c, 1.0)


def _seg_max(msgs, idx, n):
    m = jax.ops.segment_max(msgs, idx, num_segments=n)
    return jnp.where(jnp.isfinite(m), m, 0.0)


def _multiscale(x, p, src, dst, com_div):
    neigh = _seg_mean(x[src], dst, x.shape[0])
    cmean = _seg_mean(x, com_div, COM)[com_div]
    return jax.nn.relu(x @ p["Ws"] + neigh @ p["Wn"] + cmean @ p["Wc"] + p["b"])


def _matvec_body(w_ref, v_ref, o_ref):
    o_ref[...] = jnp.dot(w_ref[...], v_ref[...], preferred_element_type=jnp.float32)


def _pallas_matvec(W, v):
    M, K = W.shape
    return pl.pallas_call(
        _matvec_body,
        out_shape=jax.ShapeDtypeStruct((M, 1), jnp.float32),
        grid=(5,),
        in_specs=[
            pl.BlockSpec((M // 5, K), lambda i: (i, 0)),
            pl.BlockSpec((K, 1), lambda i: (0, 0)),
        ],
        out_specs=pl.BlockSpec((M // 5, 1), lambda i: (i, 0)),
    )(W, v)


def kernel(X, params, edge_index, com_div):
    src, dst = edge_index[0], edge_index[1]
    b = params["blocks"]
    x1 = _multiscale(X, b[0], src, dst, com_div)
    x2 = _multiscale(x1, b[1], src, dst, com_div)
    X1 = x1 + x2
    r1 = _multiscale(X1, b[2], src, dst, com_div)
    r2 = _multiscale(r1, b[3], src, dst, com_div)
    X1 = X1 + (r1 + r2)
    y1 = _multiscale(X1, b[4], src, dst, com_div)
    y2 = _multiscale(y1, b[5], src, dst, com_div)
    Xf = y1 + y2

    def _sage(x, p, aggr):
        msgs = x[src]
        if aggr == "mean":
            agg = _seg_mean(msgs, dst, x.shape[0])
        else:
            agg = _seg_max(msgs, dst, x.shape[0])
        return agg @ p["Wl"] + x @ p["Wr"] + p["b"]

    Xm = jax.nn.relu(_sage(Xf, params["sage_mean"], "mean"))
    Xx = jax.nn.relu(_sage(Xf, params["sage_max"], "max"))
    Xemb = (Xm + Xx).reshape(-1)
    h = jax.nn.relu(_pallas_matvec(params["lin"]["W"], Xemb[:, None])[:, 0] + params["lin"]["b"])
    h = jax.nn.relu(params["lin1"]["W"] @ h + params["lin1"]["b"])
    conn = jax.nn.relu(params["lin2"]["W"] @ h + params["lin2"]["b"])
    Xc = jnp.concatenate([Xf, Xemb[:, None]], axis=1)
    h2 = jax.nn.relu(Xc @ params["c1"]["W"].T + params["c1"]["b"])
    h2 = jax.nn.relu(h2 @ params["c2"]["W"].T + params["c2"]["b"])
    h2 = h2.reshape(1, -1)
    ctrl = jax.nn.relu(h2 @ params["c3"]["W"].T + params["c3"]["b"])
    conn = conn + params["phys"]
    ctrl = ctrl + params["phys"]
    return (conn, ctrl.reshape(-1))


# speedup vs baseline: 4.0237x; 4.0187x over previous
import functools

import jax
import jax.numpy as jnp
from jax import lax
from jax.experimental import pallas as pl
from jax.experimental.pallas import tpu as pltpu
from jax.experimental.pallas import tpu_sc as plsc

N = 10000
COM = 64
NPAD = 10240
NW = 32
EPT = 10240
CHUNK = 128
SUB = 8
NSUP = EPT // (CHUNK * SUB)
SLICE = NPAD // 16


def _sc_agg_body(x_hbm, src_hbm, dst_hbm, out_hbm,
                 src_v, dst_v, rows0, rows1, zbuf, acc_sh,
                 sg0, sg1, ss0, ss1):
    c = lax.axis_index("c")
    s = lax.axis_index("s")
    wid = c * 16 + s

    def _z(i, carry):
        zbuf[i] = jnp.zeros((16,), jnp.float32)
        return carry
    lax.fori_loop(0, SLICE, _z, 0)
    pltpu.sync_copy(zbuf, acc_sh.at[pl.ds(s * SLICE, SLICE)])

    pltpu.sync_copy(src_hbm.at[wid], src_v)
    pltpu.sync_copy(dst_hbm.at[wid], dst_v)

    plsc.subcore_barrier()

    rows = (rows0, rows1)
    gsem = (sg0, sg1)
    ssem = (ss0, ss1)
    pend_scatter = {0: [], 1: []}
    for g in range(NSUP):
        b = g % 2
        for cp in pend_scatter[b]:
            cp.wait()
        pend_scatter[b] = []
        gcps = []
        for k in range(SUB):
            j = g * SUB + k
            gcps.append(pltpu.async_copy(
                x_hbm.at[src_v.at[j]],
                rows[b].at[pl.ds(k * CHUNK, CHUNK)],
                gsem[b]))
        for cp in gcps:
            cp.wait()
        for k in range(SUB):
            j = g * SUB + k
            pend_scatter[b].append(pltpu.async_copy(
                rows[b].at[pl.ds(k * CHUNK, CHUNK)],
                acc_sh.at[dst_v.at[j]],
                ssem[b], add=True))
    for b in (0, 1):
        for cp in pend_scatter[b]:
            cp.wait()

    plsc.subcore_barrier()
    pltpu.sync_copy(acc_sh.at[pl.ds(s * SLICE, SLICE)],
                    out_hbm.at[c, pl.ds(s * SLICE, SLICE)])


def _sc_count_body(dst_hbm, out_hbm, dst_v, ones_v, zbuf, acc_sh, ss0):
    c = lax.axis_index("c")
    s = lax.axis_index("s")
    wid = c * 16 + s

    def _z(i, carry):
        zbuf[i] = jnp.zeros((16,), jnp.float32)
        return carry
    lax.fori_loop(0, SLICE, _z, 0)
    pltpu.sync_copy(zbuf, acc_sh.at[pl.ds(s * SLICE, SLICE)])

    def _o(i, carry):
        ones_v[i] = jnp.ones((16,), jnp.float32)
        return carry
    lax.fori_loop(0, CHUNK, _o, 0)

    pltpu.sync_copy(dst_hbm.at[wid], dst_v)
    plsc.subcore_barrier()

    for g in range(EPT // (CHUNK * SUB)):
        cps = []
        for k in range(SUB):
            j = g * SUB + k
            cps.append(pltpu.async_copy(
                ones_v, acc_sh.at[dst_v.at[j]], ss0, add=True))
        for cp in cps:
            cp.wait()

    plsc.subcore_barrier()
    pltpu.sync_copy(acc_sh.at[pl.ds(s * SLICE, SLICE)],
                    out_hbm.at[c, pl.ds(s * SLICE, SLICE)])


def _make_sc_count():
    mesh = plsc.VectorSubcoreMesh(core_axis_name="c", subcore_axis_name="s")
    return functools.partial(
        pl.kernel, mesh=mesh,
        compiler_params=pltpu.CompilerParams(use_tc_tiling_on_sc=False),
        out_type=jax.ShapeDtypeStruct((2, NPAD, 16), jnp.float32),
        scratch_types=[
            pltpu.VMEM((EPT // CHUNK, CHUNK), jnp.int32),
            pltpu.VMEM((CHUNK, 16), jnp.float32),
            pltpu.VMEM((SLICE, 16), jnp.float32),
            pltpu.VMEM_SHARED((NPAD, 16), jnp.float32),
            pltpu.SemaphoreType.DMA,
        ],
    )(_sc_count_body)


_sc_count = None


def _sc_deg_parts(dst3):
    if _DEBUG_JNP_AGG:
        dstf = dst3.reshape(-1)
        c = jax.ops.segment_sum(jnp.ones((dstf.shape[0], 16), jnp.float32),
                                dstf, num_segments=NPAD)
        return jnp.stack([c, jnp.zeros_like(c)])
    global _sc_count
    if _sc_count is None:
        _sc_count = _make_sc_count()
    return _sc_count(dst3)


def _make_sc_agg():
    mesh = plsc.VectorSubcoreMesh(core_axis_name="c", subcore_axis_name="s")
    return functools.partial(
        pl.kernel, mesh=mesh,
        compiler_params=pltpu.CompilerParams(use_tc_tiling_on_sc=False),
        out_type=jax.ShapeDtypeStruct((2, NPAD, 16), jnp.float32),
        scratch_types=[
            pltpu.VMEM((EPT // CHUNK, CHUNK), jnp.int32),
            pltpu.VMEM((EPT // CHUNK, CHUNK), jnp.int32),
            pltpu.VMEM((SUB * CHUNK, 16), jnp.float32),
            pltpu.VMEM((SUB * CHUNK, 16), jnp.float32),
            pltpu.VMEM((SLICE, 16), jnp.float32),
            pltpu.VMEM_SHARED((NPAD, 16), jnp.float32),
            pltpu.SemaphoreType.DMA,
            pltpu.SemaphoreType.DMA,
            pltpu.SemaphoreType.DMA,
            pltpu.SemaphoreType.DMA,
        ],
    )(_sc_agg_body)


_sc_agg = None


_DEBUG_JNP_AGG = False


def _sc_mean_parts(xpad, src3, dst3):
    if _DEBUG_JNP_AGG:
        srcf = src3.reshape(-1)
        dstf = dst3.reshape(-1)
        s = jax.ops.segment_sum(xpad[srcf], dstf, num_segments=NPAD)
        return jnp.stack([s, jnp.zeros_like(s)])
    global _sc_agg
    if _sc_agg is None:
        _sc_agg = _make_sc_agg()
    return _sc_agg(xpad, src3, dst3)


def _adot(a, b):
    ah = a.astype(jnp.bfloat16).astype(jnp.float32)
    al = a - ah
    bh = b.astype(jnp.bfloat16).astype(jnp.float32)
    bl = b - bh
    d = functools.partial(jnp.dot, preferred_element_type=jnp.float32)
    return d(ah, bh) + (d(ah, bl) + d(al, bh))


BN = 2048
NBLK = NPAD // BN


def _stats_body(x_ref, comt_ref, cs_ref, cc_ref):
    i = pl.program_id(0)
    oht = (comt_ref[...] == lax.broadcasted_iota(jnp.int32, (COM, BN), 0)
           ).astype(jnp.float32)
    part_cs = _adot(oht, x_ref[...])
    part_cc = jnp.sum(oht, axis=1)[:, None]

    @pl.when(i == 0)
    def _():
        cs_ref[...] = part_cs
        cc_ref[...] = part_cc

    @pl.when(i > 0)
    def _():
        cs_ref[...] += part_cs
        cc_ref[...] += part_cc


def _tc_stats(xpad, compt):
    return pl.pallas_call(
        _stats_body,
        out_shape=(jax.ShapeDtypeStruct((COM, 16), jnp.float32),
                   jax.ShapeDtypeStruct((COM, 1), jnp.float32)),
        grid=(NBLK,),
        in_specs=[
            pl.BlockSpec((BN, 16), lambda i: (i, 0)),
            pl.BlockSpec((1, BN), lambda i: (0, i)),
        ],
        out_specs=(pl.BlockSpec((COM, 16), lambda i: (0, 0)),
                   pl.BlockSpec((COM, 1), lambda i: (0, 0))),
    )(xpad, compt)


def _apply_body(x_ref, parts_ref, deg_ref, com_ref, cs_ref, cc_ref,
                ws_ref, wn_ref, wc_ref, b_ref, res_ref, o_ref):
    x = x_ref[...]
    esum = parts_ref[0] + parts_ref[1]
    deg = deg_ref[0] + deg_ref[1]
    emean = esum / jnp.maximum(deg, 1.0)
    cmeans = cs_ref[...] / jnp.maximum(cc_ref[...], 1.0)
    oh = (com_ref[...] == lax.broadcasted_iota(jnp.int32, (BN, COM), 1)
          ).astype(jnp.float32)
    cmean = _adot(oh, cmeans)
    out = jax.nn.relu(_adot(x, ws_ref[...])
                      + _adot(emean, wn_ref[...])
                      + _adot(cmean, wc_ref[...])
                      + b_ref[...])
    o_ref[...] = out + res_ref[...]


def _tc_layer(xpad, parts, dparts, comp, compt, ws, wn, wc, b, res):
    cs, cc = _tc_stats(xpad, compt)
    return pl.pallas_call(
        _apply_body,
        out_shape=jax.ShapeDtypeStruct((NPAD, 16), jnp.float32),
        grid=(NBLK,),
        in_specs=[
            pl.BlockSpec((BN, 16), lambda i: (i, 0)),
            pl.BlockSpec((2, BN, 16), lambda i: (0, i, 0)),
            pl.BlockSpec((2, BN, 1), lambda i: (0, i, 0)),
            pl.BlockSpec((BN, 1), lambda i: (i, 0)),
            pl.BlockSpec((COM, 16), lambda i: (0, 0)),
            pl.BlockSpec((COM, 1), lambda i: (0, 0)),
            pl.BlockSpec((16, 16), lambda i: (0, 0)),
            pl.BlockSpec((16, 16), lambda i: (0, 0)),
            pl.BlockSpec((16, 16), lambda i: (0, 0)),
            pl.BlockSpec((1, 16), lambda i: (0, 0)),
            pl.BlockSpec((BN, 16), lambda i: (i, 0)),
        ],
        out_specs=pl.BlockSpec((BN, 16), lambda i: (i, 0)),
    )(xpad, parts, dparts, comp, cs, cc, ws, wn, wc, b, res)


def _sage_body(xf_ref, mparts_ref, deg_ref, emax_ref, wlm_ref, wrm_ref, bm_ref,
               wlx_ref, wrx_ref, bx_ref, c1a_ref, c1b_ref, b1_ref,
               c2t_ref, b2_ref, emb_ref, h2_ref):
    xf = xf_ref[...]
    esum = mparts_ref[0] + mparts_ref[1]
    deg = deg_ref[0] + deg_ref[1]
    emean = esum / jnp.maximum(deg, 1.0)
    emax = emax_ref[...]
    xm = jax.nn.relu(_adot(emean, wlm_ref[...])
                     + _adot(xf, wrm_ref[...])
                     + bm_ref[...])
    xx = jax.nn.relu(_adot(emax, wlx_ref[...])
                     + _adot(xf, wrx_ref[...])
                     + bx_ref[...])
    emb = xm[:, 0:1] + xx[:, 0:1]
    emb_ref[...] = emb
    h2 = jax.nn.relu(_adot(xf, c1a_ref[...])
                     + _adot(emb, c1b_ref[...])
                     + b1_ref[...])
    h2_ref[...] = jax.nn.relu(
        _adot(h2, c2t_ref[...]) + b2_ref[...])


def _tc_sage(xf, mparts, dparts, emax, wlm, wrm, bm, wlx, wrx, bx,
             c1a, c1b, b1, c2t, b2):
    wspec = pl.BlockSpec((16, 16), lambda i: (0, 0))
    bspec = pl.BlockSpec((1, 16), lambda i: (0, 0))
    return pl.pallas_call(
        _sage_body,
        out_shape=(jax.ShapeDtypeStruct((NPAD, 1), jnp.float32),
                   jax.ShapeDtypeStruct((NPAD, 16), jnp.float32)),
        grid=(NBLK,),
        in_specs=[
            pl.BlockSpec((BN, 16), lambda i: (i, 0)),
            pl.BlockSpec((2, BN, 16), lambda i: (0, i, 0)),
            pl.BlockSpec((2, BN, 1), lambda i: (0, i, 0)),
            pl.BlockSpec((BN, 16), lambda i: (i, 0)),
            wspec, wspec, bspec,
            wspec, wspec, bspec,
            wspec, bspec, bspec,
            wspec, bspec,
        ],
        out_specs=(pl.BlockSpec((BN, 1), lambda i: (i, 0)),
                   pl.BlockSpec((BN, 16), lambda i: (i, 0))),
    )(xf, mparts, dparts, emax, wlm, wrm, bm, wlx, wrx, bx,
      c1a, c1b, b1, c2t, b2)


def _mv_body(w_ref, v_ref, b_ref, o_ref):
    o_ref[...] = jax.nn.relu(
        jnp.dot(w_ref[...], v_ref[...], preferred_element_type=jnp.float32)
        + b_ref[...])


def _tc_matvec_relu(W, v, b, blk):
    M, K = W.shape
    g = M // blk
    return pl.pallas_call(
        _mv_body,
        out_shape=jax.ShapeDtypeStruct((M, 1), jnp.float32),
        grid=(g,),
        in_specs=[
            pl.BlockSpec((blk, K), lambda i: (i, 0)),
            pl.BlockSpec((K, 1), lambda i: (0, 0)),
            pl.BlockSpec((blk, 1), lambda i: (i, 0)),
        ],
        out_specs=pl.BlockSpec((blk, 1), lambda i: (i, 0)),
    )(W, v, b)


def _head2_body(w1_ref, b1_ref, w2_ref, b2_ref, ph_ref, h_ref, o_ref):
    h1 = jax.nn.relu(jnp.dot(w1_ref[...], h_ref[...],
                             preferred_element_type=jnp.float32) + b1_ref[...])
    o_ref[...] = jax.nn.relu(
        jnp.dot(w2_ref[...], h1, preferred_element_type=jnp.float32)
        + b2_ref[...]) + ph_ref[...]


def _tc_head2(w1, b1, w2, b2, ph, h):
    return pl.pallas_call(
        _head2_body,
        out_shape=jax.ShapeDtypeStruct((19, 1), jnp.float32),
    )(w1, b1, w2, b2, ph, h)


def _ctrl_body(w_ref, v_ref, b_ref, ph_ref, o_ref):
    i = pl.program_id(0)
    part = jnp.dot(w_ref[...], v_ref[...], preferred_element_type=jnp.float32)

    @pl.when(i == 0)
    def _():
        o_ref[...] = part

    @pl.when(i > 0)
    def _():
        o_ref[...] += part

    @pl.when(i == pl.num_programs(0) - 1)
    def _():
        o_ref[...] = jax.nn.relu(o_ref[...] + b_ref[...]) + ph_ref[...]


def _tc_ctrl(W, v, b, ph):
    KB = 32000
    g = W.shape[1] // KB
    return pl.pallas_call(
        _ctrl_body,
        out_shape=jax.ShapeDtypeStruct((19, 1), jnp.float32),
        grid=(g,),
        in_specs=[
            pl.BlockSpec((19, KB), lambda i: (0, i)),
            pl.BlockSpec((KB, 1), lambda i: (i, 0)),
            pl.BlockSpec((19, 1), lambda i: (0, 0)),
            pl.BlockSpec((19, 1), lambda i: (0, 0)),
        ],
        out_specs=pl.BlockSpec((19, 1), lambda i: (0, 0)),
    )(W, v, b, ph)


def _pad16(W):
    din, dout = W.shape
    return jnp.zeros((16, 16), jnp.float32).at[:din, :dout].set(W)


def _padb(b):
    return jnp.zeros((1, 16), jnp.float32).at[0, :b.shape[0]].set(b)


def kernel(X, params, edge_index, com_div):
    src, dst = edge_index[0], edge_index[1]

    npadx = NPAD - N
    epad = NW * EPT - src.shape[0]
    srcp = jnp.concatenate([src, jnp.zeros((epad,), src.dtype)])
    dstp = jnp.concatenate([dst, jnp.full((epad,), NPAD - 1, dst.dtype)])
    src3 = srcp.reshape(NW, EPT // CHUNK, CHUNK).astype(jnp.int32)
    dst3 = dstp.reshape(NW, EPT // CHUNK, CHUNK).astype(jnp.int32)
    comp = jnp.concatenate([com_div.astype(jnp.int32),
                            jnp.full((npadx,), COM, jnp.int32)]).reshape(NPAD, 1)
    compt = comp.reshape(1, NPAD)

    xpad0 = jnp.zeros((NPAD, 16), jnp.float32).at[:N, :9].set(X)

    b = params["blocks"]
    zres = jnp.zeros((NPAD, 16), jnp.float32)

    degp = _sc_deg_parts(dst3)
    dparts = degp[:, :, 0:1]

    def layer(xp, p, res):
        parts = _sc_mean_parts(xp, src3, dst3)
        return _tc_layer(xp, parts, dparts, comp, compt, _pad16(p["Ws"]),
                         _pad16(p["Wn"]), _pad16(p["Wc"]), _padb(p["b"]), res)

    x1 = layer(xpad0, b[0], zres)
    X1 = layer(x1, b[1], x1)
    r1 = layer(X1, b[2], zres)
    X1b = layer(r1, b[3], X1 + r1)
    y1 = layer(X1b, b[4], zres)
    Xf = layer(y1, b[5], y1)

    mparts = _sc_mean_parts(Xf, src3, dst3)
    msgs = Xf[:N, :][src]
    emax_n = jax.ops.segment_max(msgs, dst, num_segments=N)
    emax_n = jnp.where(jnp.isfinite(emax_n), emax_n, 0.0)
    emax = jnp.zeros((NPAD, 16), jnp.float32).at[:N].set(emax_n)

    sm, sx = params["sage_mean"], params["sage_max"]

    def sage_w(p):
        wl = jnp.zeros((16, 16), jnp.float32).at[:8, 0].set(p["Wl"][:, 0])
        wr = jnp.zeros((16, 16), jnp.float32).at[:8, 0].set(p["Wr"][:, 0])
        bb = jnp.zeros((1, 16), jnp.float32).at[0, 0].set(p["b"][0])
        return wl, wr, bb

    wlm, wrm, bm = sage_w(sm)
    wlx, wrx, bx = sage_w(sx)
    c1 = params["c1"]
    c1a = jnp.zeros((16, 16), jnp.float32).at[:8, :].set(c1["W"][:, :8].T)
    c1b = c1["W"][:, 8].reshape(1, 16)
    b1 = c1["b"].reshape(1, 16)
    c2t = params["c2"]["W"].T
    b2 = params["c2"]["b"].reshape(1, 16)

    emb, h2 = _tc_sage(Xf, mparts, dparts, emax, wlm, wrm, bm, wlx, wrx, bx,
                       c1a, c1b, b1, c2t, b2)

    emb_n = emb[:N]
    h = _tc_matvec_relu(params["lin"]["W"], emb_n,
                        params["lin"]["b"].reshape(-1, 1), 200)
    conn = _tc_head2(params["lin1"]["W"], params["lin1"]["b"].reshape(-1, 1),
                     params["lin2"]["W"], params["lin2"]["b"].reshape(-1, 1),
                     params["phys"].reshape(-1, 1), h)
    h2flat = h2[:N].reshape(N * 16, 1)
    ctrl = _tc_ctrl(params["c3"]["W"], h2flat,
                    params["c3"]["b"].reshape(-1, 1),
                    params["phys"].reshape(-1, 1))
    return (conn.reshape(-1), ctrl.reshape(-1))


# speedup vs baseline: 9.3532x; 2.3245x over previous
import functools

import jax
import jax.numpy as jnp
from jax import lax
from jax.experimental import pallas as pl
from jax.experimental.pallas import tpu as pltpu
from jax.experimental.pallas import tpu_sc as plsc

N = 10000
COM = 64
NPAD = 10240
NW = 32
EPT = 10240
CHUNK = 128
SUB = 8
NSUP = EPT // (CHUNK * SUB)
SLICE = NPAD // 16


def _sc_agg_body(x_hbm, src_hbm, dst_hbm, out_hbm,
                 src_v, dst_v, rows0, rows1, zbuf, acc_sh,
                 sg0, sg1, ss0, ss1):
    c = lax.axis_index("c")
    s = lax.axis_index("s")
    wid = c * 16 + s

    def _z(i, carry):
        zbuf[i] = jnp.zeros((16,), jnp.float32)
        return carry
    lax.fori_loop(0, SLICE, _z, 0)
    pltpu.sync_copy(zbuf, acc_sh.at[pl.ds(s * SLICE, SLICE)])

    pltpu.sync_copy(src_hbm.at[wid], src_v)
    pltpu.sync_copy(dst_hbm.at[wid], dst_v)

    plsc.subcore_barrier()

    rows = (rows0, rows1)
    gsem = (sg0, sg1)
    ssem = (ss0, ss1)
    pend_scatter = {0: [], 1: []}
    for g in range(NSUP):
        b = g % 2
        for cp in pend_scatter[b]:
            cp.wait()
        pend_scatter[b] = []
        gcps = []
        for k in range(SUB):
            j = g * SUB + k
            gcps.append(pltpu.async_copy(
                x_hbm.at[src_v.at[j]],
                rows[b].at[pl.ds(k * CHUNK, CHUNK)],
                gsem[b]))
        for cp in gcps:
            cp.wait()
        for k in range(SUB):
            j = g * SUB + k
            pend_scatter[b].append(pltpu.async_copy(
                rows[b].at[pl.ds(k * CHUNK, CHUNK)],
                acc_sh.at[dst_v.at[j]],
                ssem[b], add=True))
    for b in (0, 1):
        for cp in pend_scatter[b]:
            cp.wait()

    plsc.subcore_barrier()
    pltpu.sync_copy(acc_sh.at[pl.ds(s * SLICE, SLICE)],
                    out_hbm.at[c, pl.ds(s * SLICE, SLICE)])


def _sc_count_body(dst_hbm, out_hbm, dst_v, ones_v, zbuf, acc_sh, ss0):
    c = lax.axis_index("c")
    s = lax.axis_index("s")
    wid = c * 16 + s

    def _z(i, carry):
        zbuf[i] = jnp.zeros((16,), jnp.float32)
        return carry
    lax.fori_loop(0, SLICE, _z, 0)
    pltpu.sync_copy(zbuf, acc_sh.at[pl.ds(s * SLICE, SLICE)])

    def _o(i, carry):
        ones_v[i] = jnp.ones((16,), jnp.float32)
        return carry
    lax.fori_loop(0, CHUNK, _o, 0)

    pltpu.sync_copy(dst_hbm.at[wid], dst_v)
    plsc.subcore_barrier()

    for g in range(EPT // (CHUNK * SUB)):
        cps = []
        for k in range(SUB):
            j = g * SUB + k
            cps.append(pltpu.async_copy(
                ones_v, acc_sh.at[dst_v.at[j]], ss0, add=True))
        for cp in cps:
            cp.wait()

    plsc.subcore_barrier()
    pltpu.sync_copy(acc_sh.at[pl.ds(s * SLICE, SLICE)],
                    out_hbm.at[c, pl.ds(s * SLICE, SLICE)])


def _make_sc_count():
    mesh = plsc.VectorSubcoreMesh(core_axis_name="c", subcore_axis_name="s")
    return functools.partial(
        pl.kernel, mesh=mesh,
        compiler_params=pltpu.CompilerParams(use_tc_tiling_on_sc=False),
        out_type=jax.ShapeDtypeStruct((2, NPAD, 16), jnp.float32),
        scratch_types=[
            pltpu.VMEM((EPT // CHUNK, CHUNK), jnp.int32),
            pltpu.VMEM((CHUNK, 16), jnp.float32),
            pltpu.VMEM((SLICE, 16), jnp.float32),
            pltpu.VMEM_SHARED((NPAD, 16), jnp.float32),
            pltpu.SemaphoreType.DMA,
        ],
    )(_sc_count_body)


_sc_count = None


def _sc_deg_parts(dst3):
    if _DEBUG_JNP_AGG:
        dstf = dst3.reshape(-1)
        c = jax.ops.segment_sum(jnp.ones((dstf.shape[0], 16), jnp.float32),
                                dstf, num_segments=NPAD)
        return jnp.stack([c, jnp.zeros_like(c)])
    global _sc_count
    if _sc_count is None:
        _sc_count = _make_sc_count()
    return _sc_count(dst3)


def _make_sc_agg():
    mesh = plsc.VectorSubcoreMesh(core_axis_name="c", subcore_axis_name="s")
    return functools.partial(
        pl.kernel, mesh=mesh,
        compiler_params=pltpu.CompilerParams(use_tc_tiling_on_sc=False),
        out_type=jax.ShapeDtypeStruct((2, NPAD, 16), jnp.float32),
        scratch_types=[
            pltpu.VMEM((EPT // CHUNK, CHUNK), jnp.int32),
            pltpu.VMEM((EPT // CHUNK, CHUNK), jnp.int32),
            pltpu.VMEM((SUB * CHUNK, 16), jnp.float32),
            pltpu.VMEM((SUB * CHUNK, 16), jnp.float32),
            pltpu.VMEM((SLICE, 16), jnp.float32),
            pltpu.VMEM_SHARED((NPAD, 16), jnp.float32),
            pltpu.SemaphoreType.DMA,
            pltpu.SemaphoreType.DMA,
            pltpu.SemaphoreType.DMA,
            pltpu.SemaphoreType.DMA,
        ],
    )(_sc_agg_body)


_sc_agg = None


_DEBUG_JNP_AGG = False


def _sc_mean_parts(xpad, src3, dst3):
    if _DEBUG_JNP_AGG:
        srcf = src3.reshape(-1)
        dstf = dst3.reshape(-1)
        s = jax.ops.segment_sum(xpad[srcf], dstf, num_segments=NPAD)
        return jnp.stack([s, jnp.zeros_like(s)])
    global _sc_agg
    if _sc_agg is None:
        _sc_agg = _make_sc_agg()
    return _sc_agg(xpad, src3, dst3)


NPAIR = EPT // CHUNK // 2
MROW = NPAD // 2


def _sc_max_body(x_hbm, src_hbm, dst_hbm, out_hbm,
                 src_v, dst_v, rows0, rows1, macc, kbuf, vbuf, sg0, sg1):
    c = lax.axis_index("c")
    s = lax.axis_index("s")
    wid = c * 16 + s

    def _z(i, carry):
        macc[i] = jnp.zeros((16,), jnp.float32)
        return carry
    lax.fori_loop(0, MROW, _z, 0)

    pltpu.sync_copy(src_hbm.at[wid], src_v)
    pltpu.sync_copy(dst_hbm.at[wid], dst_v)

    lanes = lax.iota(jnp.int32, 16)
    upper = lanes >= 8
    zero16 = jnp.zeros((16,), jnp.int32)
    colmin8 = lax.rem(lanes, 8)
    swapidx = lax.rem(lanes + 8, 16)

    def process(rows, j):
        for w in range(8):
            dst16 = dst_v[j, pl.ds(w * 16, 16)]
            kbuf[0] = plsc.bitcast(dst16, jnp.float32)
            for t in range(8):
                pairidx = jnp.where(upper, 2 * t + 1, 2 * t)
                revidx = jnp.where(upper, 2 * t, 2 * t + 1)
                dstp = plsc.bitcast(plsc.load_gather(kbuf, [zero16, pairidx]), jnp.int32)
                dstq = plsc.bitcast(plsc.load_gather(kbuf, [zero16, revidx]), jnp.int32)
                same = dstp == dstq
                val = plsc.load_gather(rows, [w * 16 + pairidx, colmin8])
                vbuf[0] = val
                vswap = plsc.load_gather(vbuf, [zero16, swapidx])
                val = jnp.where(same, jnp.maximum(val, vswap), val)
                wmask = jnp.logical_not(jnp.logical_and(same, upper))
                flat = dstp * 8 + colmin8
                maj = lax.shift_right_logical(flat, 4)
                mn = lax.bitwise_and(flat, 15)
                cur = plsc.load_gather(macc, [maj, mn], mask=wmask)
                plsc.store_scatter(macc, [maj, mn], jnp.maximum(cur, val),
                                   mask=wmask)

    cp0 = pltpu.async_copy(x_hbm.at[src_v.at[0]], rows0, sg0)

    def pair(g, carry):
        j0 = 2 * g
        pltpu.async_copy(x_hbm.at[src_v.at[j0 + 1]], rows1, sg1)
        pltpu.make_async_copy(x_hbm.at[src_v.at[0]], rows0, sg0).wait()
        process(rows0, j0)

        @pl.when(g < NPAIR - 1)
        def _():
            pltpu.async_copy(x_hbm.at[src_v.at[j0 + 2]], rows0, sg0)

        pltpu.make_async_copy(x_hbm.at[src_v.at[0]], rows1, sg1).wait()
        process(rows1, j0 + 1)
        return carry

    lax.fori_loop(0, NPAIR, pair, 0)

    pltpu.sync_copy(macc, out_hbm.at[wid])


def _make_sc_max():
    mesh = plsc.VectorSubcoreMesh(core_axis_name="c", subcore_axis_name="s")
    return functools.partial(
        pl.kernel, mesh=mesh,
        compiler_params=pltpu.CompilerParams(use_tc_tiling_on_sc=False,
                                             needs_layout_passes=False),
        out_type=jax.ShapeDtypeStruct((NW, MROW, 16), jnp.float32),
        scratch_types=[
            pltpu.VMEM((EPT // CHUNK, CHUNK), jnp.int32),
            pltpu.VMEM((EPT // CHUNK, CHUNK), jnp.int32),
            pltpu.VMEM((CHUNK, 16), jnp.float32),
            pltpu.VMEM((CHUNK, 16), jnp.float32),
            pltpu.VMEM((MROW, 16), jnp.float32),
            pltpu.VMEM((8, 16), jnp.float32),
            pltpu.VMEM((8, 16), jnp.float32),
            pltpu.SemaphoreType.DMA,
            pltpu.SemaphoreType.DMA,
        ],
    )(_sc_max_body)


_sc_max = None


def _sc_max_parts(xpad, src3, dst3):
    global _sc_max
    if _sc_max is None:
        _sc_max = _make_sc_max()
    return _sc_max(xpad, src3, dst3)


def _maxred_body(p_ref, o_ref):
    m = p_ref[0]
    for i in range(1, NW):
        m = jnp.maximum(m, p_ref[i])
    o_ref[...] = m


def _tc_maxreduce(planes):
    return pl.pallas_call(
        _maxred_body,
        out_shape=jax.ShapeDtypeStruct((640, 128), jnp.float32),
        grid=(8,),
        in_specs=[pl.BlockSpec((NW, 80, 128), lambda i: (0, i, 0))],
        out_specs=pl.BlockSpec((80, 128), lambda i: (i, 0)),
    )(planes)


def _adot(a, b):
    ah = a.astype(jnp.bfloat16).astype(jnp.float32)
    al = a - ah
    bh = b.astype(jnp.bfloat16).astype(jnp.float32)
    bl = b - bh
    d = functools.partial(jnp.dot, preferred_element_type=jnp.float32)
    return d(ah, bh) + (d(ah, bl) + d(al, bh))


BN = 2048
NBLK = NPAD // BN


def _stats_body(x_ref, comt_ref, cs_ref, cc_ref):
    i = pl.program_id(0)
    oht = (comt_ref[...] == lax.broadcasted_iota(jnp.int32, (COM, BN), 0)
           ).astype(jnp.float32)
    part_cs = _adot(oht, x_ref[...])
    part_cc = jnp.sum(oht, axis=1)[:, None]

    @pl.when(i == 0)
    def _():
        cs_ref[...] = part_cs
        cc_ref[...] = part_cc

    @pl.when(i > 0)
    def _():
        cs_ref[...] += part_cs
        cc_ref[...] += part_cc


def _tc_stats(xpad, compt):
    return pl.pallas_call(
        _stats_body,
        out_shape=(jax.ShapeDtypeStruct((COM, 16), jnp.float32),
                   jax.ShapeDtypeStruct((COM, 1), jnp.float32)),
        grid=(NBLK,),
        in_specs=[
            pl.BlockSpec((BN, 16), lambda i: (i, 0)),
            pl.BlockSpec((1, BN), lambda i: (0, i)),
        ],
        out_specs=(pl.BlockSpec((COM, 16), lambda i: (0, 0)),
                   pl.BlockSpec((COM, 1), lambda i: (0, 0))),
    )(xpad, compt)


def _apply_body(x_ref, parts_ref, deg_ref, com_ref, cs_ref, cc_ref,
                ws_ref, wn_ref, wc_ref, b_ref, res_ref, o_ref):
    x = x_ref[...]
    esum = parts_ref[0] + parts_ref[1]
    deg = deg_ref[0] + deg_ref[1]
    emean = esum / jnp.maximum(deg, 1.0)
    cmeans = cs_ref[...] / jnp.maximum(cc_ref[...], 1.0)
    oh = (com_ref[...] == lax.broadcasted_iota(jnp.int32, (BN, COM), 1)
          ).astype(jnp.float32)
    cmean = _adot(oh, cmeans)
    d = functools.partial(jnp.dot, preferred_element_type=jnp.float32)
    out = jax.nn.relu(d(x, ws_ref[...])
                      + d(emean, wn_ref[...])
                      + d(cmean, wc_ref[...])
                      + b_ref[...])
    o_ref[...] = out + res_ref[...]


def _tc_layer(xpad, parts, dparts, comp, compt, ws, wn, wc, b, res):
    cs, cc = _tc_stats(xpad, compt)
    return pl.pallas_call(
        _apply_body,
        out_shape=jax.ShapeDtypeStruct((NPAD, 16), jnp.float32),
        grid=(NBLK,),
        in_specs=[
            pl.BlockSpec((BN, 16), lambda i: (i, 0)),
            pl.BlockSpec((2, BN, 16), lambda i: (0, i, 0)),
            pl.BlockSpec((2, BN, 1), lambda i: (0, i, 0)),
            pl.BlockSpec((BN, 1), lambda i: (i, 0)),
            pl.BlockSpec((COM, 16), lambda i: (0, 0)),
            pl.BlockSpec((COM, 1), lambda i: (0, 0)),
            pl.BlockSpec((16, 16), lambda i: (0, 0)),
            pl.BlockSpec((16, 16), lambda i: (0, 0)),
            pl.BlockSpec((16, 16), lambda i: (0, 0)),
            pl.BlockSpec((1, 16), lambda i: (0, 0)),
            pl.BlockSpec((BN, 16), lambda i: (i, 0)),
        ],
        out_specs=pl.BlockSpec((BN, 16), lambda i: (i, 0)),
    )(xpad, parts, dparts, comp, cs, cc, ws, wn, wc, b, res)


def _ddot(a, b):
    return jnp.dot(a, b, preferred_element_type=jnp.float32)


def _sage_body(xf_ref, mparts_ref, deg_ref, emax_ref, wlm_ref, wrm_ref, bm_ref,
               wlx_ref, wrx_ref, bx_ref, c1a_ref, c1b_ref, b1_ref,
               c2t_ref, b2_ref, emb_ref, h2_ref):
    xf = xf_ref[...]
    esum = mparts_ref[0] + mparts_ref[1]
    deg = deg_ref[0] + deg_ref[1]
    emean = esum / jnp.maximum(deg, 1.0)
    emax = emax_ref[...]
    xm = jax.nn.relu(_ddot(emean, wlm_ref[...])
                     + _ddot(xf, wrm_ref[...])
                     + bm_ref[...])
    xx = jax.nn.relu(_ddot(emax, wlx_ref[...])
                     + _ddot(xf, wrx_ref[...])
                     + bx_ref[...])
    emb = xm[:, 0:1] + xx[:, 0:1]
    emb_ref[...] = emb
    h2 = jax.nn.relu(_ddot(xf, c1a_ref[...])
                     + _ddot(emb, c1b_ref[...])
                     + b1_ref[...])
    h2_ref[...] = jax.nn.relu(
        _ddot(h2, c2t_ref[...]) + b2_ref[...])


def _tc_sage(xf, mparts, dparts, emax, wlm, wrm, bm, wlx, wrx, bx,
             c1a, c1b, b1, c2t, b2):
    wspec = pl.BlockSpec((16, 16), lambda i: (0, 0))
    bspec = pl.BlockSpec((1, 16), lambda i: (0, 0))
    return pl.pallas_call(
        _sage_body,
        out_shape=(jax.ShapeDtypeStruct((NPAD, 1), jnp.float32),
                   jax.ShapeDtypeStruct((NPAD, 16), jnp.float32)),
        grid=(NBLK,),
        in_specs=[
            pl.BlockSpec((BN, 16), lambda i: (i, 0)),
            pl.BlockSpec((2, BN, 16), lambda i: (0, i, 0)),
            pl.BlockSpec((2, BN, 1), lambda i: (0, i, 0)),
            pl.BlockSpec((BN, 8), lambda i: (i, 0)),
            wspec, wspec, bspec,
            pl.BlockSpec((8, 16), lambda i: (0, 0)), wspec, bspec,
            wspec, bspec, bspec,
            wspec, bspec,
        ],
        out_specs=(pl.BlockSpec((BN, 1), lambda i: (i, 0)),
                   pl.BlockSpec((BN, 16), lambda i: (i, 0))),
    )(xf, mparts, dparts, emax, wlm, wrm, bm, wlx, wrx, bx,
      c1a, c1b, b1, c2t, b2)


def _mv_body(w_ref, v_ref, b_ref, o_ref):
    o_ref[...] = jax.nn.relu(
        jnp.dot(w_ref[...], v_ref[...], preferred_element_type=jnp.float32)
        + b_ref[...])


def _tc_matvec_relu(W, v, b, blk):
    M, K = W.shape
    g = M // blk
    return pl.pallas_call(
        _mv_body,
        out_shape=jax.ShapeDtypeStruct((M, 1), jnp.float32),
        grid=(g,),
        in_specs=[
            pl.BlockSpec((blk, K), lambda i: (i, 0)),
            pl.BlockSpec((K, 1), lambda i: (0, 0)),
            pl.BlockSpec((blk, 1), lambda i: (i, 0)),
        ],
        out_specs=pl.BlockSpec((blk, 1), lambda i: (i, 0)),
    )(W, v, b)


def _head2_body(w1_ref, b1_ref, w2_ref, b2_ref, ph_ref, h_ref, o_ref):
    h1 = jax.nn.relu(jnp.dot(w1_ref[...], h_ref[...],
                             preferred_element_type=jnp.float32) + b1_ref[...])
    o_ref[...] = jax.nn.relu(
        jnp.dot(w2_ref[...], h1, preferred_element_type=jnp.float32)
        + b2_ref[...]) + ph_ref[...]


def _tc_head2(w1, b1, w2, b2, ph, h):
    return pl.pallas_call(
        _head2_body,
        out_shape=jax.ShapeDtypeStruct((19, 1), jnp.float32),
    )(w1, b1, w2, b2, ph, h)


def _ctrl_body(w_ref, v_ref, b_ref, ph_ref, o_ref):
    i = pl.program_id(0)
    part = jnp.dot(w_ref[...], v_ref[...], preferred_element_type=jnp.float32)

    @pl.when(i == 0)
    def _():
        o_ref[...] = part

    @pl.when(i > 0)
    def _():
        o_ref[...] += part

    @pl.when(i == pl.num_programs(0) - 1)
    def _():
        o_ref[...] = jax.nn.relu(o_ref[...] + b_ref[...]) + ph_ref[...]


def _tc_ctrl(W, v, b, ph):
    KB = 32000
    g = W.shape[1] // KB
    return pl.pallas_call(
        _ctrl_body,
        out_shape=jax.ShapeDtypeStruct((19, 1), jnp.float32),
        grid=(g,),
        in_specs=[
            pl.BlockSpec((19, KB), lambda i: (0, i)),
            pl.BlockSpec((KB, 1), lambda i: (i, 0)),
            pl.BlockSpec((19, 1), lambda i: (0, 0)),
            pl.BlockSpec((19, 1), lambda i: (0, 0)),
        ],
        out_specs=pl.BlockSpec((19, 1), lambda i: (0, 0)),
    )(W, v, b, ph)


def _pad16(W):
    din, dout = W.shape
    return jnp.zeros((16, 16), jnp.float32).at[:din, :dout].set(W)


def _padb(b):
    return jnp.zeros((1, 16), jnp.float32).at[0, :b.shape[0]].set(b)


def kernel(X, params, edge_index, com_div):
    src, dst = edge_index[0], edge_index[1]

    npadx = NPAD - N
    epad = NW * EPT - src.shape[0]
    srcp = jnp.concatenate([src, jnp.zeros((epad,), src.dtype)])
    dstp = jnp.concatenate([dst, jnp.full((epad,), NPAD - 1, dst.dtype)])
    src3 = srcp.reshape(NW, EPT // CHUNK, CHUNK).astype(jnp.int32)
    dst3 = dstp.reshape(NW, EPT // CHUNK, CHUNK).astype(jnp.int32)
    comp = jnp.concatenate([com_div.astype(jnp.int32),
                            jnp.full((npadx,), COM, jnp.int32)]).reshape(NPAD, 1)
    compt = comp.reshape(1, NPAD)

    xpad0 = jnp.zeros((NPAD, 16), jnp.float32).at[:N, :9].set(X)

    b = params["blocks"]
    zres = jnp.zeros((NPAD, 16), jnp.float32)

    degp = _sc_deg_parts(dst3)
    dparts = degp[:, :, 0:1]

    def layer(xp, p, res):
        parts = _sc_mean_parts(xp, src3, dst3)
        return _tc_layer(xp, parts, dparts, comp, compt, _pad16(p["Ws"]),
                         _pad16(p["Wn"]), _pad16(p["Wc"]), _padb(p["b"]), res)

    x1 = layer(xpad0, b[0], zres)
    X1 = layer(x1, b[1], x1)
    r1 = layer(X1, b[2], zres)
    X1b = layer(r1, b[3], X1 + r1)
    y1 = layer(X1b, b[4], zres)
    Xf = layer(y1, b[5], y1)

    mparts = _sc_mean_parts(Xf, src3, dst3)
    mplanes = _sc_max_parts(Xf, src3, dst3).reshape(NW, 640, 128)
    emax = _tc_maxreduce(mplanes).reshape(NPAD, 8)

    sm, sx = params["sage_mean"], params["sage_max"]

    def sage_w(p, din):
        wl = jnp.zeros((din, 16), jnp.float32).at[:8, 0].set(p["Wl"][:, 0])
        wr = jnp.zeros((16, 16), jnp.float32).at[:8, 0].set(p["Wr"][:, 0])
        bb = jnp.zeros((1, 16), jnp.float32).at[0, 0].set(p["b"][0])
        return wl, wr, bb

    wlm, wrm, bm = sage_w(sm, 16)
    wlx, wrx, bx = sage_w(sx, 8)
    c1 = params["c1"]
    c1a = jnp.zeros((16, 16), jnp.float32).at[:8, :].set(c1["W"][:, :8].T)
    c1b = c1["W"][:, 8].reshape(1, 16)
    b1 = c1["b"].reshape(1, 16)
    c2t = params["c2"]["W"].T
    b2 = params["c2"]["b"].reshape(1, 16)

    emb, h2 = _tc_sage(Xf, mparts, dparts, emax, wlm, wrm, bm, wlx, wrx, bx,
                       c1a, c1b, b1, c2t, b2)

    emb_n = emb[:N]
    h = _tc_matvec_relu(params["lin"]["W"], emb_n,
                        params["lin"]["b"].reshape(-1, 1), 200)
    conn = _tc_head2(params["lin1"]["W"], params["lin1"]["b"].reshape(-1, 1),
                     params["lin2"]["W"], params["lin2"]["b"].reshape(-1, 1),
                     params["phys"].reshape(-1, 1), h)
    h2flat = h2[:N].reshape(N * 16, 1)
    ctrl = _tc_ctrl(params["c3"]["W"], h2flat,
                    params["c3"]["b"].reshape(-1, 1),
                    params["phys"].reshape(-1, 1))
    return (conn.reshape(-1), ctrl.reshape(-1))
